# Initial kernel scaffold; baseline (speedup 1.0000x reference)
import jax
import jax.numpy as jnp
from jax.experimental import pallas as pl


def kernel(feature, edge_list, W1, a_src1, a_dst1, b1, W2, a_src2, a_dst2, b2, Wl, bl, Wp, bp):
    raise NotImplementedError("write your pallas kernel here")


# speedup vs baseline: 44.0478x; 44.0478x over previous
import functools

import jax
import jax.numpy as jnp
from jax.experimental import pallas as pl

B, N, E = 1024, 14, 64
F_IN, HID, HEADS = 128, 256, 8
BG = 8
M = BG * N
ME = BG * E


def _gat_layer(X, Wf, a_s, a_d, b_row, Cbd, maskneg, relu):
    F = X.shape[1]
    W3 = Wf.reshape(F, HEADS, HID)
    Vsrc = (W3 * a_s[None]).sum(-1)
    Vdst = (W3 * a_d[None]).sum(-1)
    asrc = jnp.dot(X, Vsrc, preferred_element_type=jnp.float32)
    adst = jnp.dot(X, Vdst, preferred_element_type=jnp.float32)
    asrcT = asrc.T
    acc = jnp.zeros((M, HID), jnp.float32)
    for h in range(HEADS):
        Hh = jnp.dot(X, Wf[:, h * HID:(h + 1) * HID],
                     preferred_element_type=jnp.float32)
        alpha = adst[:, h:h + 1] + asrcT[h:h + 1, :]
        alpha = jnp.where(alpha >= 0, alpha, 0.2 * alpha)
        am = alpha + maskneg
        mrow = jnp.max(am, axis=1, keepdims=True)
        e = Cbd * jnp.exp(am - mrow)
        denom = jnp.sum(e, axis=1, keepdims=True)
        P = e / denom
        acc = acc + jnp.dot(P, Hh, preferred_element_type=jnp.float32)
    out = acc * (1.0 / HEADS) + b_row
    if relu:
        out = jnp.maximum(out, 0.0)
    return out


def _block_kernel(x_ref, src_ref, dst_ref,
                  w1_ref, as1_ref, ad1_ref, b1_ref,
                  w2_ref, as2_ref, ad2_ref, b2_ref,
                  wl_ref, bl_ref, wp_ref, bp_ref, out_ref):
    X = x_ref[...].reshape(M, F_IN)

    goff = (jax.lax.broadcasted_iota(jnp.int32, (ME, 1), 0) // E) * N
    gsrc = src_ref[...] + goff
    gdst = dst_ref[...] + goff
    col = jax.lax.broadcasted_iota(jnp.int32, (ME, M), 1)
    S2 = (gsrc == col).astype(jnp.float32)
    D2 = (gdst == col).astype(jnp.float32)
    Cbd = jax.lax.dot_general(D2, S2, (((0,), (0,)), ((), ())),
                              preferred_element_type=jnp.float32)
    r_i = jax.lax.broadcasted_iota(jnp.int32, (M, M), 0)
    c_i = jax.lax.broadcasted_iota(jnp.int32, (M, M), 1)
    Cbd = Cbd + (r_i == c_i).astype(jnp.float32)
    maskneg = jnp.where(Cbd > 0, 0.0, -1e30)

    X2 = _gat_layer(X, w1_ref[...], as1_ref[...], ad1_ref[...], b1_ref[...],
                    Cbd, maskneg, relu=True)
    H2 = _gat_layer(X2, w2_ref[...], as2_ref[...], ad2_ref[...], b2_ref[...],
                    Cbd, maskneg, relu=False)

    H2r = H2.reshape(BG, N, HID)
    z = jnp.zeros((BG, HID // 2), jnp.float32) + bl_ref[...]
    for n in range(N):
        z = z + jnp.dot(H2r[:, n, :], wl_ref[n * HID:(n + 1) * HID, :],
                        preferred_element_type=jnp.float32)
    p = jnp.dot(z, wp_ref[...], preferred_element_type=jnp.float32) + bp_ref[...]
    out_ref[...] = jax.nn.sigmoid(p)


@jax.jit
def kernel(feature, edge_list, W1, a_src1, a_dst1, b1,
           W2, a_src2, a_dst2, b2, Wl, bl, Wp, bp):
    ei = edge_list.astype(jnp.int32)
    src = ei[:, :, 0].reshape(B * E, 1)
    dst = ei[:, :, 1].reshape(B * E, 1)
    b1r = b1.reshape(1, HID)
    b2r = b2.reshape(1, HID)
    blr = bl.reshape(1, HID // 2)
    bpr = bp.reshape(1, 1)

    const = lambda i: (0, 0)
    grid = B // BG
    out = pl.pallas_call(
        _block_kernel,
        grid=(grid,),
        in_specs=[
            pl.BlockSpec((BG, N, F_IN), lambda i: (i, 0, 0)),
            pl.BlockSpec((ME, 1), lambda i: (i, 0)),
            pl.BlockSpec((ME, 1), lambda i: (i, 0)),
            pl.BlockSpec((F_IN, HEADS * HID), const),
            pl.BlockSpec((HEADS, HID), const),
            pl.BlockSpec((HEADS, HID), const),
            pl.BlockSpec((1, HID), const),
            pl.BlockSpec((HID, HEADS * HID), const),
            pl.BlockSpec((HEADS, HID), const),
            pl.BlockSpec((HEADS, HID), const),
            pl.BlockSpec((1, HID), const),
            pl.BlockSpec((HID * N, HID // 2), const),
            pl.BlockSpec((1, HID // 2), const),
            pl.BlockSpec((HID // 2, 1), const),
            pl.BlockSpec((1, 1), const),
        ],
        out_specs=pl.BlockSpec((BG, 1), lambda i: (i, 0)),
        out_shape=jax.ShapeDtypeStruct((B, 1), jnp.float32),
    )(feature, src, dst, W1, a_src1, a_dst1, b1r,
      W2, a_src2, a_dst2, b2r, Wl, blr, Wp, bpr)
    return out


# speedup vs baseline: 58.3586x; 1.3249x over previous
import functools

import jax
import jax.numpy as jnp
from jax.experimental import pallas as pl
from jax.experimental.pallas import tpu as pltpu

B, N, E = 1024, 14, 64
F_IN, HID, HEADS = 128, 256, 8
BG = 8
M = BG * N
ME = BG * E


def _gat_layer(X, Wf, Vsrc, Vdst, b_row, Cbd, maskneg, relu):
    Hall = jnp.dot(X, Wf, preferred_element_type=jnp.float32)
    asrc = jnp.dot(X, Vsrc, preferred_element_type=jnp.float32)
    adst = jnp.dot(X, Vdst, preferred_element_type=jnp.float32)
    asrcT = asrc.T
    acc = jnp.zeros((M, HID), jnp.float32)
    for h in range(HEADS):
        Hh = Hall[:, h * HID:(h + 1) * HID]
        alpha = adst[:, h:h + 1] + asrcT[h:h + 1, :]
        alpha = jnp.where(alpha >= 0, alpha, 0.2 * alpha)
        am = alpha + maskneg
        mrow = jnp.max(am, axis=1, keepdims=True)
        e = Cbd * jnp.exp(am - mrow)
        denom = jnp.sum(e, axis=1, keepdims=True)
        P = e / denom
        acc = acc + jnp.dot(P, Hh, preferred_element_type=jnp.float32)
    out = acc * (1.0 / HEADS) + b_row
    if relu:
        out = jnp.maximum(out, 0.0)
    return out


def _block_kernel(x_ref, src_ref, dst_ref,
                  w1_ref, as1_ref, ad1_ref, b1_ref,
                  w2_ref, as2_ref, ad2_ref, b2_ref,
                  wl_ref, bl_ref, wp_ref, bp_ref, out_ref,
                  vs1_ref, vd1_ref, vs2_ref, vd2_ref):
    @pl.when(pl.program_id(0) == 0)
    def _fold():
        W31 = w1_ref[...].reshape(F_IN, HEADS, HID)
        vs1_ref[...] = (W31 * as1_ref[...][None]).sum(-1)
        vd1_ref[...] = (W31 * ad1_ref[...][None]).sum(-1)
        W32 = w2_ref[...].reshape(HID, HEADS, HID)
        vs2_ref[...] = (W32 * as2_ref[...][None]).sum(-1)
        vd2_ref[...] = (W32 * ad2_ref[...][None]).sum(-1)

    X = x_ref[...].reshape(M, F_IN)

    goff = (jax.lax.broadcasted_iota(jnp.int32, (ME, 1), 0) // E) * N
    gsrc = src_ref[...] + goff
    gdst = dst_ref[...] + goff
    col = jax.lax.broadcasted_iota(jnp.int32, (ME, M), 1)
    S2 = (gsrc == col).astype(jnp.float32)
    D2 = (gdst == col).astype(jnp.float32)
    Cbd = jax.lax.dot_general(D2, S2, (((0,), (0,)), ((), ())),
                              preferred_element_type=jnp.float32)
    r_i = jax.lax.broadcasted_iota(jnp.int32, (M, M), 0)
    c_i = jax.lax.broadcasted_iota(jnp.int32, (M, M), 1)
    Cbd = Cbd + (r_i == c_i).astype(jnp.float32)
    maskneg = jnp.where(Cbd > 0, 0.0, -1e30)

    X2 = _gat_layer(X, w1_ref[...], vs1_ref[...], vd1_ref[...], b1_ref[...],
                    Cbd, maskneg, relu=True)
    H2 = _gat_layer(X2, w2_ref[...], vs2_ref[...], vd2_ref[...], b2_ref[...],
                    Cbd, maskneg, relu=False)

    H2r = H2.reshape(BG, N, HID)
    z = jnp.zeros((BG, HID // 2), jnp.float32) + bl_ref[...]
    for n in range(N):
        z = z + jnp.dot(H2r[:, n, :], wl_ref[n * HID:(n + 1) * HID, :],
                        preferred_element_type=jnp.float32)
    p = jnp.dot(z, wp_ref[...], preferred_element_type=jnp.float32) + bp_ref[...]
    out_ref[...] = jax.nn.sigmoid(p)


@jax.jit
def kernel(feature, edge_list, W1, a_src1, a_dst1, b1,
           W2, a_src2, a_dst2, b2, Wl, bl, Wp, bp):
    ei = edge_list.astype(jnp.int32)
    src = ei[:, :, 0].reshape(B * E, 1)
    dst = ei[:, :, 1].reshape(B * E, 1)
    b1r = b1.reshape(1, HID)
    b2r = b2.reshape(1, HID)
    blr = bl.reshape(1, HID // 2)
    bpr = bp.reshape(1, 1)

    const = lambda i: (0, 0)
    grid = B // BG
    out = pl.pallas_call(
        _block_kernel,
        grid=(grid,),
        in_specs=[
            pl.BlockSpec((BG, N, F_IN), lambda i: (i, 0, 0)),
            pl.BlockSpec((ME, 1), lambda i: (i, 0)),
            pl.BlockSpec((ME, 1), lambda i: (i, 0)),
            pl.BlockSpec((F_IN, HEADS * HID), const),
            pl.BlockSpec((HEADS, HID), const),
            pl.BlockSpec((HEADS, HID), const),
            pl.BlockSpec((1, HID), const),
            pl.BlockSpec((HID, HEADS * HID), const),
            pl.BlockSpec((HEADS, HID), const),
            pl.BlockSpec((HEADS, HID), const),
            pl.BlockSpec((1, HID), const),
            pl.BlockSpec((HID * N, HID // 2), const),
            pl.BlockSpec((1, HID // 2), const),
            pl.BlockSpec((HID // 2, 1), const),
            pl.BlockSpec((1, 1), const),
        ],
        out_specs=pl.BlockSpec((BG, 1), lambda i: (i, 0)),
        out_shape=jax.ShapeDtypeStruct((B, 1), jnp.float32),
        scratch_shapes=[
            pltpu.VMEM((F_IN, HEADS), jnp.float32),
            pltpu.VMEM((F_IN, HEADS), jnp.float32),
            pltpu.VMEM((HID, HEADS), jnp.float32),
            pltpu.VMEM((HID, HEADS), jnp.float32),
        ],
    )(feature, src, dst, W1, a_src1, a_dst1, b1r,
      W2, a_src2, a_dst2, b2r, Wl, blr, Wp, bpr)
    return out


# speedup vs baseline: 104.9082x; 1.7976x over previous
import functools

import jax
import jax.numpy as jnp
from jax.experimental import pallas as pl
from jax.experimental.pallas import tpu as pltpu

B, N, E = 1024, 14, 64
F_IN, HID, HEADS = 128, 256, 8
BG = 8
M = BG * N
ME = BG * E


def _gat_layer(X, Wf, Vsrc, Vdst, b_row, Cbd, maskneg, relu):
    Hall = jnp.dot(X, Wf, preferred_element_type=jnp.float32)
    asrc = jnp.dot(X, Vsrc, preferred_element_type=jnp.float32)
    adst = jnp.dot(X, Vdst, preferred_element_type=jnp.float32)
    asrcT = asrc.T
    acc = jnp.zeros((M, HID), jnp.float32)
    for h in range(HEADS):
        Hh = Hall[:, h * HID:(h + 1) * HID]
        alpha = adst[:, h:h + 1] + asrcT[h:h + 1, :]
        alpha = jnp.where(alpha >= 0, alpha, 0.2 * alpha)
        e = Cbd * jnp.exp(jnp.minimum(alpha + maskneg, 60.0))
        rden = 1.0 / jnp.sum(e, axis=1, keepdims=True)
        acc = acc + jnp.dot(e, Hh, preferred_element_type=jnp.float32) * rden
    out = acc * (1.0 / HEADS) + b_row
    if relu:
        out = jnp.maximum(out, 0.0)
    return out


def _block_kernel(x_ref, src_ref, dst_ref,
                  w1_ref, as1_ref, ad1_ref, b1_ref,
                  w2_ref, as2_ref, ad2_ref, b2_ref,
                  wl_ref, bl_ref, wp_ref, bp_ref, out_ref,
                  vs1_ref, vd1_ref, vs2_ref, vd2_ref):
    @pl.when(pl.program_id(0) == 0)
    def _fold():
        W31 = w1_ref[...].reshape(F_IN, HEADS, HID)
        vs1_ref[...] = (W31 * as1_ref[...][None]).sum(-1)
        vd1_ref[...] = (W31 * ad1_ref[...][None]).sum(-1)
        W32 = w2_ref[...].reshape(HID, HEADS, HID)
        vs2_ref[...] = (W32 * as2_ref[...][None]).sum(-1)
        vd2_ref[...] = (W32 * ad2_ref[...][None]).sum(-1)

    X = x_ref[...].reshape(M, F_IN)

    goff = (jax.lax.broadcasted_iota(jnp.int32, (ME, 1), 0) // E) * N
    gsrc = src_ref[...] + goff
    col = jax.lax.broadcasted_iota(jnp.int32, (ME, M), 1)
    S2 = (gsrc == col).astype(jnp.float32)
    goff_r = (jax.lax.broadcasted_iota(jnp.int32, (1, ME), 1) // E) * N
    gdst_r = dst_ref[...].reshape(1, ME) + goff_r
    row = jax.lax.broadcasted_iota(jnp.int32, (M, ME), 0)
    D2T = (gdst_r == row).astype(jnp.float32)
    Cbd = jnp.dot(D2T, S2, preferred_element_type=jnp.float32)
    r_i = jax.lax.broadcasted_iota(jnp.int32, (M, M), 0)
    c_i = jax.lax.broadcasted_iota(jnp.int32, (M, M), 1)
    Cbd = Cbd + (r_i == c_i).astype(jnp.float32)
    maskneg = jnp.where(Cbd > 0, 0.0, -1e30)

    X2 = _gat_layer(X, w1_ref[...], vs1_ref[...], vd1_ref[...], b1_ref[...],
                    Cbd, maskneg, relu=True)
    H2 = _gat_layer(X2, w2_ref[...], vs2_ref[...], vd2_ref[...], b2_ref[...],
                    Cbd, maskneg, relu=False)

    H2r = H2.reshape(BG, N, HID)
    z = jnp.zeros((BG, HID // 2), jnp.float32) + bl_ref[...]
    for n in range(N):
        z = z + jnp.dot(H2r[:, n, :], wl_ref[n * HID:(n + 1) * HID, :],
                        preferred_element_type=jnp.float32)
    p = jnp.dot(z, wp_ref[...], preferred_element_type=jnp.float32) + bp_ref[...]
    out_ref[...] = jax.nn.sigmoid(p)


@jax.jit
def kernel(feature, edge_list, W1, a_src1, a_dst1, b1,
           W2, a_src2, a_dst2, b2, Wl, bl, Wp, bp):
    ei = edge_list.astype(jnp.int32)
    src = ei[:, :, 0].reshape(B * E, 1)
    dst = ei[:, :, 1].reshape(B // BG, 1, ME)
    b1r = b1.reshape(1, HID)
    b2r = b2.reshape(1, HID)
    blr = bl.reshape(1, HID // 2)
    bpr = bp.reshape(1, 1)

    const = lambda i: (0, 0)
    grid = B // BG
    out = pl.pallas_call(
        _block_kernel,
        grid=(grid,),
        in_specs=[
            pl.BlockSpec((BG, N, F_IN), lambda i: (i, 0, 0)),
            pl.BlockSpec((ME, 1), lambda i: (i, 0)),
            pl.BlockSpec((1, 1, ME), lambda i: (i, 0, 0)),
            pl.BlockSpec((F_IN, HEADS * HID), const),
            pl.BlockSpec((HEADS, HID), const),
            pl.BlockSpec((HEADS, HID), const),
            pl.BlockSpec((1, HID), const),
            pl.BlockSpec((HID, HEADS * HID), const),
            pl.BlockSpec((HEADS, HID), const),
            pl.BlockSpec((HEADS, HID), const),
            pl.BlockSpec((1, HID), const),
            pl.BlockSpec((HID * N, HID // 2), const),
            pl.BlockSpec((1, HID // 2), const),
            pl.BlockSpec((HID // 2, 1), const),
            pl.BlockSpec((1, 1), const),
        ],
        out_specs=pl.BlockSpec((BG, 1), lambda i: (i, 0)),
        out_shape=jax.ShapeDtypeStruct((B, 1), jnp.float32),
        scratch_shapes=[
            pltpu.VMEM((F_IN, HEADS), jnp.float32),
            pltpu.VMEM((F_IN, HEADS), jnp.float32),
            pltpu.VMEM((HID, HEADS), jnp.float32),
            pltpu.VMEM((HID, HEADS), jnp.float32),
        ],
    )(feature, src, dst, W1, a_src1, a_dst1, b1r,
      W2, a_src2, a_dst2, b2r, Wl, blr, Wp, bpr)
    return out
